# baseline (speedup 1.0000x reference)
import functools

import jax
import jax.numpy as jnp
from jax import lax
from jax.experimental import pallas as pl
from jax.experimental.pallas import tpu as pltpu
from jax.experimental.pallas import tpu_sc as plsc

N = 50000
E = 800000
EPAD = 802816
ACC_ROWS = 51200
TRASH = N
BATCH = 128
JPS = 14
RPT = 3200


def _mesh():
    return plsc.VectorSubcoreMesh(core_axis_name="c", subcore_axis_name="s")


_SC_PARAMS = pltpu.CompilerParams(use_tc_tiling_on_sc=False)


def _deg_body(src_ref, dst_ref, zer, ones, od32_ref, id32_ref, id8_ref,
              acc, sidx, ones_v, dbuf, wbuf, wbuf8, dsem):
    c = lax.axis_index("c")
    s = lax.axis_index("s")
    pltpu.sync_copy(zer, acc.at[pl.ds(s * RPT, RPT)])
    pltpu.sync_copy(ones, ones_v)
    plsc.subcore_barrier()

    def hist(idx_hbm):
        @pl.loop(0, 28)
        def _slab(sl):
            pltpu.sync_copy(idx_hbm.at[pl.ds(s * 392 + sl * JPS, JPS)], sidx)

            @pl.loop(0, JPS)
            def _j(j):
                pltpu.async_copy(ones_v, acc.at[sidx.at[j]], dsem, add=True)

            @pl.loop(0, JPS)
            def _jw(j):
                pltpu.make_async_copy(ones_v, acc.at[sidx.at[j]], dsem).wait()

    @pl.when(c == 0)
    def _():
        hist(src_ref)

    @pl.when(c == 1)
    def _():
        hist(dst_ref)

    plsc.subcore_barrier()

    pltpu.sync_copy(acc.at[pl.ds(s * RPT, RPT)], dbuf)

    def expand_to(out_ref):
        @pl.loop(0, 4)
        def _chunk(k):
            @pl.loop(0, 50)
            def _n16(n16):
                v = dbuf[pl.ds(k * 800 + n16 * 16, 16)]
                for lane in range(16):
                    v16 = jnp.full((16,), v[lane], jnp.float32)
                    wbuf[n16 * 16 + lane, pl.ds(0, 16)] = v16
                    wbuf[n16 * 16 + lane, pl.ds(16, 16)] = v16
            pltpu.sync_copy(wbuf, out_ref.at[pl.ds(s * RPT + k * 800, 800)])

    def expand8_to(out_ref):
        lane = lax.iota(jnp.int32, 16)

        @pl.loop(0, 4)
        def _chunk(k):
            @pl.loop(0, 50)
            def _n16(n16):
                v = dbuf[pl.ds(k * 800 + n16 * 16, 16)]
                for l in range(8):
                    lo = jnp.full((16,), v[2 * l], jnp.float32)
                    hi = jnp.full((16,), v[2 * l + 1], jnp.float32)
                    wbuf8[n16 * 8 + l, :] = jnp.where(lane < 8, lo, hi)
            pltpu.sync_copy(
                wbuf8, out_ref.at[pl.ds(s * 1600 + k * 400, 400)])

    @pl.when(c == 0)
    def _():
        expand_to(od32_ref)

    @pl.when(c == 1)
    def _():
        expand_to(id32_ref)
        expand8_to(id8_ref)


def _degrees(src_s, dst_s, zer1, ones1):
    f = pl.kernel(
        _deg_body,
        out_type=(
            jax.ShapeDtypeStruct((ACC_ROWS, 32), jnp.float32),
            jax.ShapeDtypeStruct((ACC_ROWS, 32), jnp.float32),
            jax.ShapeDtypeStruct((ACC_ROWS // 2, 16), jnp.float32),
        ),
        mesh=_mesh(),
        compiler_params=_SC_PARAMS,
        scratch_types=[
            pltpu.VMEM_SHARED((ACC_ROWS,), jnp.float32),
            pltpu.VMEM((JPS, BATCH), jnp.int32),
            pltpu.VMEM((BATCH,), jnp.float32),
            pltpu.VMEM((RPT,), jnp.float32),
            pltpu.VMEM((800, 32), jnp.float32),
            pltpu.VMEM((400, 16), jnp.float32),
            pltpu.SemaphoreType.DMA,
        ],
    )
    return f(src_s, dst_s, zer1, ones1)


def _agg_body(depth, slots, edge_split, nslabs, t0_ref, t1_ref, src_ref,
              dst_ref, zer, p0_ref, p1_ref, acc, sidx, didx, rows, gsem, ssem):
    c = lax.axis_index("c")
    s = lax.axis_index("s")
    pltpu.sync_copy(zer, acc.at[pl.ds(s * RPT, RPT)])
    plsc.subcore_barrier()

    wid = c * 16 + s if edge_split else s
    rows_per_w = nslabs * JPS

    def run(table):
        @pl.loop(0, nslabs)
        def _slab(sl):
            base = wid * rows_per_w + sl * JPS
            pltpu.sync_copy(src_ref.at[pl.ds(base, JPS)], sidx)
            pltpu.sync_copy(dst_ref.at[pl.ds(base, JPS)], didx)
            for d in range(depth):
                pltpu.async_copy(table.at[sidx.at[d]], rows.at[d], gsem.at[d])
            for j in range(JPS):
                sj = j % slots
                pltpu.make_async_copy(
                    table.at[sidx.at[j]], rows.at[sj], gsem.at[sj]).wait()
                pltpu.async_copy(rows.at[sj], acc.at[didx.at[j]],
                                 ssem.at[sj], add=True)
                jn = j + depth
                if jn < JPS:
                    snj = jn % slots
                    if jn - slots >= 0:
                        pltpu.make_async_copy(
                            rows.at[snj], acc.at[didx.at[jn - slots]],
                            ssem.at[snj]).wait()
                    pltpu.async_copy(table.at[sidx.at[jn]], rows.at[snj],
                                     gsem.at[snj])
            for j in range(max(0, JPS - slots), JPS):
                pltpu.make_async_copy(rows.at[j % slots], acc.at[didx.at[j]],
                                      ssem.at[j % slots]).wait()

    if edge_split:
        run(t0_ref)
    else:
        @pl.when(c == 0)
        def _():
            run(t0_ref)

        @pl.when(c == 1)
        def _():
            run(t1_ref)

    plsc.subcore_barrier()
    sl_ = pl.ds(s * RPT, RPT)

    @pl.when(c == 0)
    def _():
        pltpu.sync_copy(acc.at[sl_], p0_ref.at[sl_])

    @pl.when(c == 1)
    def _():
        pltpu.sync_copy(acc.at[sl_], p1_ref.at[sl_])


def _aggregate(table0, table1, src_r, dst_r, zer, feat, edge_split):
    nslabs = 14 if edge_split else 28
    depth = 3
    slots = 6
    body = functools.partial(_agg_body, depth, slots, edge_split, nslabs)
    f = pl.kernel(
        body,
        out_type=(
            jax.ShapeDtypeStruct((ACC_ROWS, feat), jnp.float32),
            jax.ShapeDtypeStruct((ACC_ROWS, feat), jnp.float32),
        ),
        mesh=_mesh(),
        compiler_params=_SC_PARAMS,
        scratch_types=[
            pltpu.VMEM_SHARED((ACC_ROWS, feat), jnp.float32),
            pltpu.VMEM((JPS, BATCH), jnp.int32),
            pltpu.VMEM((JPS, BATCH), jnp.int32),
            pltpu.VMEM((slots, BATCH, feat), jnp.float32),
            pltpu.SemaphoreType.DMA((slots,)),
            pltpu.SemaphoreType.DMA((slots,)),
        ],
    )
    return f(table0, table1, src_r, dst_r, zer)


def _blk128(nb):
    return pl.BlockSpec((nb, 128), lambda i: (i, 0))


def _wspec():
    return pl.BlockSpec((128, 128), lambda i: (0, 0))


def _bspec():
    return pl.BlockSpec((1, 128), lambda i: (0, 0))


def _inv(deg):
    return lax.rsqrt(jnp.maximum(deg, 1.0))


def _k1_body(x_ref, od_ref, xn_ref):
    xn_ref[...] = x_ref[...] * _inv(od_ref[...][:12500])


def _k1(xpp, od32p):
    return pl.pallas_call(
        _k1_body,
        grid=(1,),
        in_specs=[_blk128(12500), _blk128(12800)],
        out_specs=_blk128(12500),
        out_shape=jax.ShapeDtypeStruct((12500, 128), jnp.float32),
    )(xpp, od32p)


def _k3_body(p0_ref, p1_ref, od_ref, id_ref, wa_ref, wb_ref, ba_ref, bb_ref,
             ha_ref, hb_ref):
    agg = p0_ref[...] + p1_ref[...]
    iin = _inv(id_ref[...])
    iout = _inv(od_ref[...])
    ma = jnp.dot(agg, wa_ref[...], preferred_element_type=jnp.float32)
    mb = jnp.dot(agg, wb_ref[...], preferred_element_type=jnp.float32)
    ha_ref[...] = jnp.tanh(ma * iin + ba_ref[...]) * iout
    hb_ref[...] = jnp.tanh(mb * iin + bb_ref[...]) * iout


def _k3(p0p, p1p, od32p, id32p, w1a, w1b, b1a, b1b):
    return pl.pallas_call(
        _k3_body,
        grid=(2,),
        in_specs=[_blk128(6400), _blk128(6400), _blk128(6400), _blk128(6400),
                  _wspec(), _wspec(), _bspec(), _bspec()],
        out_specs=(_blk128(6400), _blk128(6400)),
        out_shape=(jax.ShapeDtypeStruct((12800, 128), jnp.float32),
                   jax.ShapeDtypeStruct((12800, 128), jnp.float32)),
    )(p0p, p1p, od32p, id32p, w1a, w1b, b1a, b1b)


def _k5_body(a_ref, b_ref, od_ref, id_ref, waa_ref, wba_ref, wab_ref,
             wbb_ref, w3a_ref, w3b_ref, b2a_ref, b2b_ref, g_ref):
    a = a_ref[...]
    b = b_ref[...]
    iin = _inv(id_ref[...])
    iout = _inv(od_ref[...])
    h2a = jnp.tanh((jnp.dot(a, waa_ref[...], preferred_element_type=jnp.float32)
                    + jnp.dot(b, wba_ref[...], preferred_element_type=jnp.float32))
                   * iin + b2a_ref[...]) * iout
    h2b = jnp.tanh((jnp.dot(a, wab_ref[...], preferred_element_type=jnp.float32)
                    + jnp.dot(b, wbb_ref[...], preferred_element_type=jnp.float32))
                   * iin + b2b_ref[...]) * iout
    g_ref[...] = (jnp.dot(h2a, w3a_ref[...], preferred_element_type=jnp.float32)
                  + jnp.dot(h2b, w3b_ref[...], preferred_element_type=jnp.float32))


def _k5(a2ap, a2bp, od32p, id32p, waa, wba, wab, wbb, w3a, w3b, b2a, b2b):
    return pl.pallas_call(
        _k5_body,
        grid=(2,),
        in_specs=[_blk128(6400), _blk128(6400), _blk128(6400), _blk128(6400),
                  _wspec(), _wspec(), _wspec(), _wspec(),
                  _wspec(), _wspec(), _bspec(), _bspec()],
        out_specs=_blk128(6400),
        out_shape=jax.ShapeDtypeStruct((12800, 128), jnp.float32),
    )(a2ap, a2bp, od32p, id32p, waa, wba, wab, wbb, w3a, w3b, b2a, b2b)


def _k7_body(q0_ref, q1_ref, id_ref, b3_ref, o_ref):
    o_ref[...] = (q0_ref[...] + q1_ref[...]) * _inv(id_ref[...]) + b3_ref[...]


def _k7(q0p, q1p, id8p, b3w):
    return pl.pallas_call(
        _k7_body,
        grid=(2,),
        in_specs=[_blk128(1600), _blk128(1600), _blk128(1600), _bspec()],
        out_specs=_blk128(1600),
        out_shape=jax.ShapeDtypeStruct((3200, 128), jnp.float32),
    )(q0p, q1p, id8p, b3w)


def _bd4(m):
    return jnp.kron(jnp.eye(4, dtype=jnp.float32), m)


def kernel(x, edge_index, W1, b1, W2, b2, W3, b3):
    src = edge_index[0].astype(jnp.int32)
    dst = edge_index[1].astype(jnp.int32)

    npad = EPAD - E
    src_g = jnp.concatenate([src, jnp.zeros((npad,), jnp.int32)]).reshape(6272, 128)
    src_s = jnp.concatenate([src, jnp.full((npad,), TRASH, jnp.int32)]).reshape(6272, 128)
    dst_s = jnp.concatenate([dst, jnp.full((npad,), TRASH, jnp.int32)]).reshape(6272, 128)

    zer1 = jnp.zeros((RPT,), jnp.float32)
    zer32 = jnp.zeros((RPT, 32), jnp.float32)
    ones1 = jnp.ones((BATCH,), jnp.float32)

    od32, id32, id8 = _degrees(src_s, dst_s, zer1, ones1)
    od32p = od32.reshape(12800, 128)
    id32p = id32.reshape(12800, 128)
    id8p = id8.reshape(3200, 128)

    xpp = x.reshape(12500, 128)
    xnp = _k1(xpp, od32p)

    xn_t = xnp.reshape(N, 32)
    p0, p1 = _aggregate(xn_t, xn_t, src_g, dst_s, zer32, 32, True)

    w1a = _bd4(W1[:, :32])
    w1b = _bd4(W1[:, 32:])
    b1a = jnp.tile(b1[:32], 4).reshape(1, 128)
    b1b = jnp.tile(b1[32:], 4).reshape(1, 128)
    h1ap, h1bp = _k3(p0.reshape(12800, 128), p1.reshape(12800, 128),
                     od32p, id32p, w1a, w1b, b1a, b1b)

    a2a, a2b = _aggregate(h1ap.reshape(ACC_ROWS, 32), h1bp.reshape(ACC_ROWS, 32),
                          src_g, dst_s, zer32, 32, False)

    w3wide = jnp.pad(W3, ((0, 0), (0, 30)))
    waa = _bd4(W2[:32, :32])
    wba = _bd4(W2[32:, :32])
    wab = _bd4(W2[:32, 32:])
    wbb = _bd4(W2[32:, 32:])
    w3a = _bd4(w3wide[:32])
    w3b = _bd4(w3wide[32:])
    b2a = jnp.tile(b2[:32], 4).reshape(1, 128)
    b2b = jnp.tile(b2[32:], 4).reshape(1, 128)
    gp = _k5(a2a.reshape(12800, 128), a2b.reshape(12800, 128),
             od32p, id32p, waa, wba, wab, wbb, w3a, w3b, b2a, b2b)

    g8 = gp.reshape(12800, 4, 32)[:, :, :8].reshape(ACC_ROWS, 8)
    zer8 = jnp.zeros((RPT, 8), jnp.float32)
    q0, q1 = _aggregate(g8, g8, src_g, dst_s, zer8, 8, True)

    b3w = jnp.tile(jnp.pad(b3, (0, 6)), 16).reshape(1, 128)
    o = _k7(q0.reshape(3200, 128), q1.reshape(3200, 128), id8p, b3w)
    return o.reshape(3200, 16, 8)[:, :, :2].reshape(ACC_ROWS, 2)[:N]


# speedup vs baseline: 1.1676x; 1.1676x over previous
import functools

import jax
import jax.numpy as jnp
from jax import lax
from jax.experimental import pallas as pl
from jax.experimental.pallas import tpu as pltpu
from jax.experimental.pallas import tpu_sc as plsc

N = 50000
E = 800000
EPAD = 802816
ACC_ROWS = 51200
TRASH = N
BATCH = 128
JPS = 28
RPT = 3200


def _mesh():
    return plsc.VectorSubcoreMesh(core_axis_name="c", subcore_axis_name="s")


_SC_PARAMS = pltpu.CompilerParams(use_tc_tiling_on_sc=False)


def _deg_body(src_ref, dst_ref, zer, ones, od32_ref, id32_ref, id8_ref,
              acc, sidx, ones_v, dbuf, wbuf, wbuf8, dsem):
    c = lax.axis_index("c")
    s = lax.axis_index("s")
    pltpu.sync_copy(zer, acc.at[pl.ds(s * RPT, RPT)])
    pltpu.sync_copy(ones, ones_v)
    plsc.subcore_barrier()

    def hist(idx_hbm):
        @pl.loop(0, 14)
        def _slab(sl):
            pltpu.sync_copy(idx_hbm.at[pl.ds(s * 392 + sl * JPS, JPS)], sidx)

            @pl.loop(0, JPS)
            def _j(j):
                pltpu.async_copy(ones_v, acc.at[sidx.at[j]], dsem, add=True)

            @pl.loop(0, JPS)
            def _jw(j):
                pltpu.make_async_copy(ones_v, acc.at[sidx.at[j]], dsem).wait()

    @pl.when(c == 0)
    def _():
        hist(src_ref)

    @pl.when(c == 1)
    def _():
        hist(dst_ref)

    plsc.subcore_barrier()

    pltpu.sync_copy(acc.at[pl.ds(s * RPT, RPT)], dbuf)

    def expand_to(out_ref):
        @pl.loop(0, 4)
        def _chunk(k):
            @pl.loop(0, 50)
            def _n16(n16):
                v = dbuf[pl.ds(k * 800 + n16 * 16, 16)]
                for lane in range(16):
                    v16 = jnp.full((16,), v[lane], jnp.float32)
                    wbuf[n16 * 16 + lane, pl.ds(0, 16)] = v16
                    wbuf[n16 * 16 + lane, pl.ds(16, 16)] = v16
            pltpu.sync_copy(wbuf, out_ref.at[pl.ds(s * RPT + k * 800, 800)])

    def expand8_to(out_ref):
        lane = lax.iota(jnp.int32, 16)

        @pl.loop(0, 4)
        def _chunk(k):
            @pl.loop(0, 50)
            def _n16(n16):
                v = dbuf[pl.ds(k * 800 + n16 * 16, 16)]
                for l in range(8):
                    lo = jnp.full((16,), v[2 * l], jnp.float32)
                    hi = jnp.full((16,), v[2 * l + 1], jnp.float32)
                    wbuf8[n16 * 8 + l, :] = jnp.where(lane < 8, lo, hi)
            pltpu.sync_copy(
                wbuf8, out_ref.at[pl.ds(s * 1600 + k * 400, 400)])

    @pl.when(c == 0)
    def _():
        expand_to(od32_ref)

    @pl.when(c == 1)
    def _():
        expand_to(id32_ref)
        expand8_to(id8_ref)


def _degrees(src_s, dst_s, zer1, ones1):
    f = pl.kernel(
        _deg_body,
        out_type=(
            jax.ShapeDtypeStruct((ACC_ROWS, 32), jnp.float32),
            jax.ShapeDtypeStruct((ACC_ROWS, 32), jnp.float32),
            jax.ShapeDtypeStruct((ACC_ROWS // 2, 16), jnp.float32),
        ),
        mesh=_mesh(),
        compiler_params=_SC_PARAMS,
        scratch_types=[
            pltpu.VMEM_SHARED((ACC_ROWS,), jnp.float32),
            pltpu.VMEM((JPS, BATCH), jnp.int32),
            pltpu.VMEM((BATCH,), jnp.float32),
            pltpu.VMEM((RPT,), jnp.float32),
            pltpu.VMEM((800, 32), jnp.float32),
            pltpu.VMEM((400, 16), jnp.float32),
            pltpu.SemaphoreType.DMA,
        ],
    )
    return f(src_s, dst_s, zer1, ones1)


def _agg_body(depth, edge_split, nslabs, t0_ref, t1_ref, src_ref, dst_ref,
              zer, p0_ref, p1_ref, acc, sidx, didx, rows, gsem):
    c = lax.axis_index("c")
    s = lax.axis_index("s")
    pltpu.sync_copy(zer, acc.at[pl.ds(s * RPT, RPT)])
    plsc.subcore_barrier()

    wid = c * 16 + s if edge_split else s
    rows_per_w = nslabs * JPS

    def run(table):
        @pl.loop(0, nslabs)
        def _slab(sl):
            base = wid * rows_per_w + sl * JPS
            pltpu.sync_copy(src_ref.at[pl.ds(base, JPS)], sidx)
            pltpu.sync_copy(dst_ref.at[pl.ds(base, JPS)], didx)
            for d in range(depth):
                pltpu.async_copy(table.at[sidx.at[d]], rows.at[d], gsem.at[d])

            @pl.loop(0, JPS // depth)
            def _jo(jo):
                for d in range(depth):
                    j = jo * depth + d
                    pltpu.make_async_copy(
                        table.at[sidx.at[j]], rows.at[d], gsem.at[d]).wait()
                    pltpu.sync_copy(rows.at[d], acc.at[didx.at[j]], add=True)

                    @pl.when(j + depth < JPS)
                    def _():
                        pltpu.async_copy(table.at[sidx.at[j + depth]],
                                         rows.at[d], gsem.at[d])

    if edge_split:
        run(t0_ref)
    else:
        @pl.when(c == 0)
        def _():
            run(t0_ref)

        @pl.when(c == 1)
        def _():
            run(t1_ref)

    plsc.subcore_barrier()
    sl_ = pl.ds(s * RPT, RPT)

    @pl.when(c == 0)
    def _():
        pltpu.sync_copy(acc.at[sl_], p0_ref.at[sl_])

    @pl.when(c == 1)
    def _():
        pltpu.sync_copy(acc.at[sl_], p1_ref.at[sl_])


def _aggregate(table0, table1, src_r, dst_r, zer, feat, edge_split):
    nslabs = 7 if edge_split else 14
    depth = 4 if feat == 32 else 14
    body = functools.partial(_agg_body, depth, edge_split, nslabs)
    f = pl.kernel(
        body,
        out_type=(
            jax.ShapeDtypeStruct((ACC_ROWS, feat), jnp.float32),
            jax.ShapeDtypeStruct((ACC_ROWS, feat), jnp.float32),
        ),
        mesh=_mesh(),
        compiler_params=_SC_PARAMS,
        scratch_types=[
            pltpu.VMEM_SHARED((ACC_ROWS, feat), jnp.float32),
            pltpu.VMEM((JPS, BATCH), jnp.int32),
            pltpu.VMEM((JPS, BATCH), jnp.int32),
            pltpu.VMEM((depth, BATCH, feat), jnp.float32),
            pltpu.SemaphoreType.DMA((depth,)),
        ],
    )
    return f(table0, table1, src_r, dst_r, zer)


def _blk128(nb):
    return pl.BlockSpec((nb, 128), lambda i: (i, 0))


def _wspec():
    return pl.BlockSpec((128, 128), lambda i: (0, 0))


def _bspec():
    return pl.BlockSpec((1, 128), lambda i: (0, 0))


def _inv(deg):
    return lax.rsqrt(jnp.maximum(deg, 1.0))


def _k1_body(x_ref, od_ref, xn_ref):
    xn_ref[...] = x_ref[...] * _inv(od_ref[...][:12500])


def _k1(xpp, od32p):
    return pl.pallas_call(
        _k1_body,
        grid=(1,),
        in_specs=[_blk128(12500), _blk128(12800)],
        out_specs=_blk128(12500),
        out_shape=jax.ShapeDtypeStruct((12500, 128), jnp.float32),
    )(xpp, od32p)


def _k3_body(p0_ref, p1_ref, od_ref, id_ref, wa_ref, wb_ref, ba_ref, bb_ref,
             ha_ref, hb_ref):
    agg = p0_ref[...] + p1_ref[...]
    iin = _inv(id_ref[...])
    iout = _inv(od_ref[...])
    ma = jnp.dot(agg, wa_ref[...], preferred_element_type=jnp.float32)
    mb = jnp.dot(agg, wb_ref[...], preferred_element_type=jnp.float32)
    ha_ref[...] = jnp.tanh(ma * iin + ba_ref[...]) * iout
    hb_ref[...] = jnp.tanh(mb * iin + bb_ref[...]) * iout


def _k3(p0p, p1p, od32p, id32p, w1a, w1b, b1a, b1b):
    return pl.pallas_call(
        _k3_body,
        grid=(2,),
        in_specs=[_blk128(6400), _blk128(6400), _blk128(6400), _blk128(6400),
                  _wspec(), _wspec(), _bspec(), _bspec()],
        out_specs=(_blk128(6400), _blk128(6400)),
        out_shape=(jax.ShapeDtypeStruct((12800, 128), jnp.float32),
                   jax.ShapeDtypeStruct((12800, 128), jnp.float32)),
    )(p0p, p1p, od32p, id32p, w1a, w1b, b1a, b1b)


def _k5_body(a_ref, b_ref, od_ref, id_ref, waa_ref, wba_ref, wab_ref,
             wbb_ref, w3a_ref, w3b_ref, b2a_ref, b2b_ref, g_ref):
    a = a_ref[...]
    b = b_ref[...]
    iin = _inv(id_ref[...])
    iout = _inv(od_ref[...])
    h2a = jnp.tanh((jnp.dot(a, waa_ref[...], preferred_element_type=jnp.float32)
                    + jnp.dot(b, wba_ref[...], preferred_element_type=jnp.float32))
                   * iin + b2a_ref[...]) * iout
    h2b = jnp.tanh((jnp.dot(a, wab_ref[...], preferred_element_type=jnp.float32)
                    + jnp.dot(b, wbb_ref[...], preferred_element_type=jnp.float32))
                   * iin + b2b_ref[...]) * iout
    g_ref[...] = (jnp.dot(h2a, w3a_ref[...], preferred_element_type=jnp.float32)
                  + jnp.dot(h2b, w3b_ref[...], preferred_element_type=jnp.float32))


def _k5(a2ap, a2bp, od32p, id32p, waa, wba, wab, wbb, w3a, w3b, b2a, b2b):
    return pl.pallas_call(
        _k5_body,
        grid=(2,),
        in_specs=[_blk128(6400), _blk128(6400), _blk128(6400), _blk128(6400),
                  _wspec(), _wspec(), _wspec(), _wspec(),
                  _wspec(), _wspec(), _bspec(), _bspec()],
        out_specs=_blk128(6400),
        out_shape=jax.ShapeDtypeStruct((12800, 128), jnp.float32),
    )(a2ap, a2bp, od32p, id32p, waa, wba, wab, wbb, w3a, w3b, b2a, b2b)


def _k7_body(q0_ref, q1_ref, id_ref, b3_ref, o_ref):
    o_ref[...] = (q0_ref[...] + q1_ref[...]) * _inv(id_ref[...]) + b3_ref[...]


def _k7(q0p, q1p, id8p, b3w):
    return pl.pallas_call(
        _k7_body,
        grid=(2,),
        in_specs=[_blk128(1600), _blk128(1600), _blk128(1600), _bspec()],
        out_specs=_blk128(1600),
        out_shape=jax.ShapeDtypeStruct((3200, 128), jnp.float32),
    )(q0p, q1p, id8p, b3w)


def _bd4(m):
    return jnp.kron(jnp.eye(4, dtype=jnp.float32), m)


def kernel(x, edge_index, W1, b1, W2, b2, W3, b3):
    src = edge_index[0].astype(jnp.int32)
    dst = edge_index[1].astype(jnp.int32)

    npad = EPAD - E
    src_g = jnp.concatenate([src, jnp.zeros((npad,), jnp.int32)]).reshape(6272, 128)
    src_s = jnp.concatenate([src, jnp.full((npad,), TRASH, jnp.int32)]).reshape(6272, 128)
    dst_s = jnp.concatenate([dst, jnp.full((npad,), TRASH, jnp.int32)]).reshape(6272, 128)

    zer1 = jnp.zeros((RPT,), jnp.float32)
    zer32 = jnp.zeros((RPT, 32), jnp.float32)
    ones1 = jnp.ones((BATCH,), jnp.float32)

    od32, id32, id8 = _degrees(src_s, dst_s, zer1, ones1)
    od32p = od32.reshape(12800, 128)
    id32p = id32.reshape(12800, 128)
    id8p = id8.reshape(3200, 128)

    xpp = x.reshape(12500, 128)
    xnp = _k1(xpp, od32p)

    xn_t = xnp.reshape(N, 32)
    p0, p1 = _aggregate(xn_t, xn_t, src_g, dst_s, zer32, 32, True)

    w1a = _bd4(W1[:, :32])
    w1b = _bd4(W1[:, 32:])
    b1a = jnp.tile(b1[:32], 4).reshape(1, 128)
    b1b = jnp.tile(b1[32:], 4).reshape(1, 128)
    h1ap, h1bp = _k3(p0.reshape(12800, 128), p1.reshape(12800, 128),
                     od32p, id32p, w1a, w1b, b1a, b1b)

    a2a, a2b = _aggregate(h1ap.reshape(ACC_ROWS, 32), h1bp.reshape(ACC_ROWS, 32),
                          src_g, dst_s, zer32, 32, False)

    w3wide = jnp.pad(W3, ((0, 0), (0, 30)))
    waa = _bd4(W2[:32, :32])
    wba = _bd4(W2[32:, :32])
    wab = _bd4(W2[:32, 32:])
    wbb = _bd4(W2[32:, 32:])
    w3a = _bd4(w3wide[:32])
    w3b = _bd4(w3wide[32:])
    b2a = jnp.tile(b2[:32], 4).reshape(1, 128)
    b2b = jnp.tile(b2[32:], 4).reshape(1, 128)
    gp = _k5(a2a.reshape(12800, 128), a2b.reshape(12800, 128),
             od32p, id32p, waa, wba, wab, wbb, w3a, w3b, b2a, b2b)

    g8 = gp.reshape(12800, 4, 32)[:, :, :8].reshape(ACC_ROWS, 8)
    zer8 = jnp.zeros((RPT, 8), jnp.float32)
    q0, q1 = _aggregate(g8, g8, src_g, dst_s, zer8, 8, True)

    b3w = jnp.tile(jnp.pad(b3, (0, 6)), 16).reshape(1, 128)
    o = _k7(q0.reshape(3200, 128), q1.reshape(3200, 128), id8p, b3w)
    return o.reshape(3200, 16, 8)[:, :, :2].reshape(ACC_ROWS, 2)[:N]


# speedup vs baseline: 1.2155x; 1.0410x over previous
import functools

import jax
import jax.numpy as jnp
from jax import lax
from jax.experimental import pallas as pl
from jax.experimental.pallas import tpu as pltpu
from jax.experimental.pallas import tpu_sc as plsc

N = 50000
E = 800000
EPAD = 802816
ACC_ROWS = 51200
TRASH = N
BATCH = 128
JPS = 28
RPT = 3200


def _mesh():
    return plsc.VectorSubcoreMesh(core_axis_name="c", subcore_axis_name="s")


_SC_PARAMS = pltpu.CompilerParams(use_tc_tiling_on_sc=False)


def _deg_body(edge_ref, padt, zer, ones, od32_ref, id32_ref, id8_ref,
              acc, sidx, ones_v, dbuf, wbuf, wbuf8, dsem):
    c = lax.axis_index("c")
    s = lax.axis_index("s")
    pltpu.sync_copy(zer, acc.at[pl.ds(s * RPT, RPT)])
    pltpu.sync_copy(ones, ones_v)
    plsc.subcore_barrier()

    def hist(row):
        @pl.loop(0, 14)
        def _slab(sl):
            base = s * 392 + sl * JPS
            is_last = jnp.logical_and(s == 15, sl == 13)

            @pl.when(jnp.logical_not(is_last))
            def _():
                pltpu.sync_copy(edge_ref.at[row, pl.ds(base, JPS)], sidx)

            @pl.when(is_last)
            def _():
                pltpu.sync_copy(edge_ref.at[row, pl.ds(6244, 6)],
                                sidx.at[pl.ds(0, 6)])
                pltpu.sync_copy(padt, sidx.at[pl.ds(6, 22)])

            @pl.loop(0, JPS)
            def _j(j):
                pltpu.async_copy(ones_v, acc.at[sidx.at[j]], dsem, add=True)

            @pl.loop(0, JPS)
            def _jw(j):
                pltpu.make_async_copy(ones_v, acc.at[sidx.at[j]], dsem).wait()

    @pl.when(c == 0)
    def _():
        hist(0)

    @pl.when(c == 1)
    def _():
        hist(1)

    plsc.subcore_barrier()

    pltpu.sync_copy(acc.at[pl.ds(s * RPT, RPT)], dbuf)

    def expand_to(out_ref):
        @pl.loop(0, 4)
        def _chunk(k):
            @pl.loop(0, 50)
            def _n16(n16):
                v = dbuf[pl.ds(k * 800 + n16 * 16, 16)]
                for lane in range(16):
                    v16 = jnp.full((16,), v[lane], jnp.float32)
                    wbuf[n16 * 16 + lane, pl.ds(0, 16)] = v16
                    wbuf[n16 * 16 + lane, pl.ds(16, 16)] = v16
            pltpu.sync_copy(wbuf, out_ref.at[pl.ds(s * RPT + k * 800, 800)])

    def expand8_to(out_ref):
        lane = lax.iota(jnp.int32, 16)

        @pl.loop(0, 4)
        def _chunk(k):
            @pl.loop(0, 50)
            def _n16(n16):
                v = dbuf[pl.ds(k * 800 + n16 * 16, 16)]
                for l in range(8):
                    lo = jnp.full((16,), v[2 * l], jnp.float32)
                    hi = jnp.full((16,), v[2 * l + 1], jnp.float32)
                    wbuf8[n16 * 8 + l, :] = jnp.where(lane < 8, lo, hi)
            pltpu.sync_copy(
                wbuf8, out_ref.at[pl.ds(s * 1600 + k * 400, 400)])

    @pl.when(c == 0)
    def _():
        expand_to(od32_ref)

    @pl.when(c == 1)
    def _():
        expand_to(id32_ref)
        expand8_to(id8_ref)


def _degrees(edge_r, padt, zer1, ones1):
    f = pl.kernel(
        _deg_body,
        out_type=(
            jax.ShapeDtypeStruct((ACC_ROWS, 32), jnp.float32),
            jax.ShapeDtypeStruct((ACC_ROWS, 32), jnp.float32),
            jax.ShapeDtypeStruct((ACC_ROWS // 2, 16), jnp.float32),
        ),
        mesh=_mesh(),
        compiler_params=_SC_PARAMS,
        scratch_types=[
            pltpu.VMEM_SHARED((ACC_ROWS,), jnp.float32),
            pltpu.VMEM((JPS, BATCH), jnp.int32),
            pltpu.VMEM((BATCH,), jnp.float32),
            pltpu.VMEM((RPT,), jnp.float32),
            pltpu.VMEM((800, 32), jnp.float32),
            pltpu.VMEM((400, 16), jnp.float32),
            pltpu.SemaphoreType.DMA,
        ],
    )
    return f(edge_r, padt, zer1, ones1)


def _agg_body(depth, edge_split, nslabs, t0_ref, t1_ref, edge_ref, spad,
              dpad, zer, p0_ref, p1_ref, acc, sidx, didx, rows, gsem):
    c = lax.axis_index("c")
    s = lax.axis_index("s")
    pltpu.sync_copy(zer, acc.at[pl.ds(s * RPT, RPT)])
    plsc.subcore_barrier()

    wid = c * 16 + s if edge_split else s
    rows_per_w = nslabs * JPS

    lastw = 31 if edge_split else 15

    def run(table):
        @pl.loop(0, nslabs)
        def _slab(sl):
            base = wid * rows_per_w + sl * JPS
            is_last = jnp.logical_and(wid == lastw, sl == nslabs - 1)

            @pl.when(jnp.logical_not(is_last))
            def _():
                pltpu.sync_copy(edge_ref.at[0, pl.ds(base, JPS)], sidx)
                pltpu.sync_copy(edge_ref.at[1, pl.ds(base, JPS)], didx)

            @pl.when(is_last)
            def _():
                pltpu.sync_copy(edge_ref.at[0, pl.ds(6244, 6)],
                                sidx.at[pl.ds(0, 6)])
                pltpu.sync_copy(spad, sidx.at[pl.ds(6, 22)])
                pltpu.sync_copy(edge_ref.at[1, pl.ds(6244, 6)],
                                didx.at[pl.ds(0, 6)])
                pltpu.sync_copy(dpad, didx.at[pl.ds(6, 22)])
            for d in range(depth):
                pltpu.async_copy(table.at[sidx.at[d]], rows.at[d], gsem.at[d])

            @pl.loop(0, JPS // depth)
            def _jo(jo):
                for d in range(depth):
                    j = jo * depth + d
                    pltpu.make_async_copy(
                        table.at[sidx.at[j]], rows.at[d], gsem.at[d]).wait()
                    pltpu.sync_copy(rows.at[d], acc.at[didx.at[j]], add=True)

                    @pl.when(j + depth < JPS)
                    def _():
                        pltpu.async_copy(table.at[sidx.at[j + depth]],
                                         rows.at[d], gsem.at[d])

    if edge_split:
        run(t0_ref)
    else:
        @pl.when(c == 0)
        def _():
            run(t0_ref)

        @pl.when(c == 1)
        def _():
            run(t1_ref)

    plsc.subcore_barrier()
    sl_ = pl.ds(s * RPT, RPT)

    @pl.when(c == 0)
    def _():
        pltpu.sync_copy(acc.at[sl_], p0_ref.at[sl_])

    @pl.when(c == 1)
    def _():
        pltpu.sync_copy(acc.at[sl_], p1_ref.at[sl_])


def _aggregate(table0, table1, edge_r, spad, dpad, zer, feat, edge_split):
    nslabs = 7 if edge_split else 14
    depth = 4 if feat == 32 else 14
    body = functools.partial(_agg_body, depth, edge_split, nslabs)
    f = pl.kernel(
        body,
        out_type=(
            jax.ShapeDtypeStruct((ACC_ROWS, feat), jnp.float32),
            jax.ShapeDtypeStruct((ACC_ROWS, feat), jnp.float32),
        ),
        mesh=_mesh(),
        compiler_params=_SC_PARAMS,
        scratch_types=[
            pltpu.VMEM_SHARED((ACC_ROWS, feat), jnp.float32),
            pltpu.VMEM((JPS, BATCH), jnp.int32),
            pltpu.VMEM((JPS, BATCH), jnp.int32),
            pltpu.VMEM((depth, BATCH, feat), jnp.float32),
            pltpu.SemaphoreType.DMA((depth,)),
        ],
    )
    return f(table0, table1, edge_r, spad, dpad, zer)


def _blk128(nb):
    return pl.BlockSpec((nb, 128), lambda i: (i, 0))


def _wspec():
    return pl.BlockSpec((128, 128), lambda i: (0, 0))


def _bspec():
    return pl.BlockSpec((1, 128), lambda i: (0, 0))


def _inv(deg):
    return lax.rsqrt(jnp.maximum(deg, 1.0))


def _k1_body(x_ref, od_ref, xn_ref):
    xn_ref[...] = x_ref[...] * _inv(od_ref[...][:12500])


def _k1(xpp, od32p):
    return pl.pallas_call(
        _k1_body,
        grid=(1,),
        in_specs=[_blk128(12500), _blk128(12800)],
        out_specs=_blk128(12500),
        out_shape=jax.ShapeDtypeStruct((12500, 128), jnp.float32),
    )(xpp, od32p)


def _k3_body(p0_ref, p1_ref, od_ref, id_ref, wa_ref, wb_ref, ba_ref, bb_ref,
             ha_ref, hb_ref):
    agg = p0_ref[...] + p1_ref[...]
    iin = _inv(id_ref[...])
    iout = _inv(od_ref[...])
    ma = jnp.dot(agg, wa_ref[...], preferred_element_type=jnp.float32)
    mb = jnp.dot(agg, wb_ref[...], preferred_element_type=jnp.float32)
    ha_ref[...] = jnp.tanh(ma * iin + ba_ref[...]) * iout
    hb_ref[...] = jnp.tanh(mb * iin + bb_ref[...]) * iout


def _k3(p0p, p1p, od32p, id32p, w1a, w1b, b1a, b1b):
    return pl.pallas_call(
        _k3_body,
        grid=(2,),
        in_specs=[_blk128(6400), _blk128(6400), _blk128(6400), _blk128(6400),
                  _wspec(), _wspec(), _bspec(), _bspec()],
        out_specs=(_blk128(6400), _blk128(6400)),
        out_shape=(jax.ShapeDtypeStruct((12800, 128), jnp.float32),
                   jax.ShapeDtypeStruct((12800, 128), jnp.float32)),
    )(p0p, p1p, od32p, id32p, w1a, w1b, b1a, b1b)


def _k5_body(a_ref, b_ref, od_ref, id_ref, waa_ref, wba_ref, wab_ref,
             wbb_ref, w3a_ref, w3b_ref, b2a_ref, b2b_ref, g_ref):
    a = a_ref[...]
    b = b_ref[...]
    iin = _inv(id_ref[...])
    iout = _inv(od_ref[...])
    h2a = jnp.tanh((jnp.dot(a, waa_ref[...], preferred_element_type=jnp.float32)
                    + jnp.dot(b, wba_ref[...], preferred_element_type=jnp.float32))
                   * iin + b2a_ref[...]) * iout
    h2b = jnp.tanh((jnp.dot(a, wab_ref[...], preferred_element_type=jnp.float32)
                    + jnp.dot(b, wbb_ref[...], preferred_element_type=jnp.float32))
                   * iin + b2b_ref[...]) * iout
    g_ref[...] = (jnp.dot(h2a, w3a_ref[...], preferred_element_type=jnp.float32)
                  + jnp.dot(h2b, w3b_ref[...], preferred_element_type=jnp.float32))


def _k5(a2ap, a2bp, od32p, id32p, waa, wba, wab, wbb, w3a, w3b, b2a, b2b):
    return pl.pallas_call(
        _k5_body,
        grid=(2,),
        in_specs=[_blk128(6400), _blk128(6400), _blk128(6400), _blk128(6400),
                  _wspec(), _wspec(), _wspec(), _wspec(),
                  _wspec(), _wspec(), _bspec(), _bspec()],
        out_specs=_blk128(6400),
        out_shape=jax.ShapeDtypeStruct((12800, 128), jnp.float32),
    )(a2ap, a2bp, od32p, id32p, waa, wba, wab, wbb, w3a, w3b, b2a, b2b)


def _k7_body(q0_ref, q1_ref, id_ref, b3_ref, o_ref):
    o_ref[...] = (q0_ref[...] + q1_ref[...]) * _inv(id_ref[...]) + b3_ref[...]


def _k7(q0p, q1p, id8p, b3w):
    return pl.pallas_call(
        _k7_body,
        grid=(2,),
        in_specs=[_blk128(1600), _blk128(1600), _blk128(1600), _bspec()],
        out_specs=_blk128(1600),
        out_shape=jax.ShapeDtypeStruct((3200, 128), jnp.float32),
    )(q0p, q1p, id8p, b3w)


def _bd4(m):
    return jnp.kron(jnp.eye(4, dtype=jnp.float32), m)


def kernel(x, edge_index, W1, b1, W2, b2, W3, b3):
    edge_r = edge_index.astype(jnp.int32).reshape(2, 6250, 128)
    pad0 = jnp.zeros((22, 128), jnp.int32)
    padt = jnp.full((22, 128), TRASH, jnp.int32)

    zer1 = jnp.zeros((RPT,), jnp.float32)
    zer32 = jnp.zeros((RPT, 32), jnp.float32)
    ones1 = jnp.ones((BATCH,), jnp.float32)

    od32, id32, id8 = _degrees(edge_r, padt, zer1, ones1)
    od32p = od32.reshape(12800, 128)
    id32p = id32.reshape(12800, 128)
    id8p = id8.reshape(3200, 128)

    xpp = x.reshape(12500, 128)
    xnp = _k1(xpp, od32p)

    xn_t = xnp.reshape(N, 32)
    p0, p1 = _aggregate(xn_t, xn_t, edge_r, pad0, padt, zer32, 32, True)

    w1a = _bd4(W1[:, :32])
    w1b = _bd4(W1[:, 32:])
    b1a = jnp.tile(b1[:32], 4).reshape(1, 128)
    b1b = jnp.tile(b1[32:], 4).reshape(1, 128)
    h1ap, h1bp = _k3(p0.reshape(12800, 128), p1.reshape(12800, 128),
                     od32p, id32p, w1a, w1b, b1a, b1b)

    a2a, a2b = _aggregate(h1ap.reshape(ACC_ROWS, 32), h1bp.reshape(ACC_ROWS, 32),
                          edge_r, pad0, padt, zer32, 32, False)

    w3wide = jnp.pad(W3, ((0, 0), (0, 30)))
    waa = _bd4(W2[:32, :32])
    wba = _bd4(W2[32:, :32])
    wab = _bd4(W2[:32, 32:])
    wbb = _bd4(W2[32:, 32:])
    w3a = _bd4(w3wide[:32])
    w3b = _bd4(w3wide[32:])
    b2a = jnp.tile(b2[:32], 4).reshape(1, 128)
    b2b = jnp.tile(b2[32:], 4).reshape(1, 128)
    gp = _k5(a2a.reshape(12800, 128), a2b.reshape(12800, 128),
             od32p, id32p, waa, wba, wab, wbb, w3a, w3b, b2a, b2b)

    g8 = gp.reshape(12800, 4, 32)[:, :, :8].reshape(ACC_ROWS, 8)
    zer8 = jnp.zeros((RPT, 8), jnp.float32)
    q0, q1 = _aggregate(g8, g8, edge_r, pad0, padt, zer8, 8, True)

    b3w = jnp.tile(jnp.pad(b3, (0, 6)), 16).reshape(1, 128)
    o = _k7(q0.reshape(3200, 128), q1.reshape(3200, 128), id8p, b3w)
    return o.reshape(3200, 16, 8)[:, :, :2].reshape(ACC_ROWS, 2)[:N]


# speedup vs baseline: 1.2438x; 1.0232x over previous
import functools

import jax
import jax.numpy as jnp
from jax import lax
from jax.experimental import pallas as pl
from jax.experimental.pallas import tpu as pltpu
from jax.experimental.pallas import tpu_sc as plsc

N = 50000
E = 800000
EPAD = 802816
ACC_ROWS = 51200
TRASH = N
BATCH = 128
JPS = 28
RPT = 3200


def _mesh():
    return plsc.VectorSubcoreMesh(core_axis_name="c", subcore_axis_name="s")


_SC_PARAMS = pltpu.CompilerParams(use_tc_tiling_on_sc=False)


def _deg_body(edge_ref, padt, zer, ones, od32_ref, id32_ref, id8_ref,
              acc, sidx, ones_v, dbuf, wbuf, wbuf8, dsem):
    c = lax.axis_index("c")
    s = lax.axis_index("s")
    pltpu.sync_copy(zer, acc.at[pl.ds(s * RPT, RPT)])
    pltpu.sync_copy(ones, ones_v)
    plsc.subcore_barrier()

    def hist(row):
        @pl.loop(0, 14)
        def _slab(sl):
            base = s * 392 + sl * JPS
            is_last = jnp.logical_and(s == 15, sl == 13)

            @pl.when(jnp.logical_not(is_last))
            def _():
                pltpu.sync_copy(edge_ref.at[row, pl.ds(base, JPS)], sidx)

            @pl.when(is_last)
            def _():
                pltpu.sync_copy(edge_ref.at[row, pl.ds(6244, 6)],
                                sidx.at[pl.ds(0, 6)])
                pltpu.sync_copy(padt, sidx.at[pl.ds(6, 22)])

            @pl.loop(0, JPS)
            def _j(j):
                pltpu.async_copy(ones_v, acc.at[sidx.at[j]], dsem, add=True)

            @pl.loop(0, JPS)
            def _jw(j):
                pltpu.make_async_copy(ones_v, acc.at[sidx.at[j]], dsem).wait()

    @pl.when(c == 0)
    def _():
        hist(0)

    @pl.when(c == 1)
    def _():
        hist(1)

    plsc.subcore_barrier()

    pltpu.sync_copy(acc.at[pl.ds(s * RPT, RPT)], dbuf)

    def expand_to(out_ref):
        @pl.loop(0, 4)
        def _chunk(k):
            @pl.loop(0, 50)
            def _n16(n16):
                v = dbuf[pl.ds(k * 800 + n16 * 16, 16)]
                for lane in range(16):
                    v16 = jnp.full((16,), v[lane], jnp.float32)
                    wbuf[n16 * 16 + lane, pl.ds(0, 16)] = v16
                    wbuf[n16 * 16 + lane, pl.ds(16, 16)] = v16
            pltpu.sync_copy(wbuf, out_ref.at[pl.ds(s * RPT + k * 800, 800)])

    def expand8_to(out_ref):
        lane = lax.iota(jnp.int32, 16)

        @pl.loop(0, 4)
        def _chunk(k):
            @pl.loop(0, 50)
            def _n16(n16):
                v = dbuf[pl.ds(k * 800 + n16 * 16, 16)]
                for l in range(8):
                    lo = jnp.full((16,), v[2 * l], jnp.float32)
                    hi = jnp.full((16,), v[2 * l + 1], jnp.float32)
                    wbuf8[n16 * 8 + l, :] = jnp.where(lane < 8, lo, hi)
            pltpu.sync_copy(
                wbuf8, out_ref.at[pl.ds(s * 1600 + k * 400, 400)])

    @pl.when(c == 0)
    def _():
        expand_to(od32_ref)

    @pl.when(c == 1)
    def _():
        expand_to(id32_ref)
        expand8_to(id8_ref)


def _degrees(edge_r, padt, zer1, ones1):
    f = pl.kernel(
        _deg_body,
        out_type=(
            jax.ShapeDtypeStruct((ACC_ROWS, 32), jnp.float32),
            jax.ShapeDtypeStruct((ACC_ROWS, 32), jnp.float32),
            jax.ShapeDtypeStruct((ACC_ROWS // 2, 16), jnp.float32),
        ),
        mesh=_mesh(),
        compiler_params=_SC_PARAMS,
        scratch_types=[
            pltpu.VMEM_SHARED((ACC_ROWS,), jnp.float32),
            pltpu.VMEM((JPS, BATCH), jnp.int32),
            pltpu.VMEM((BATCH,), jnp.float32),
            pltpu.VMEM((RPT,), jnp.float32),
            pltpu.VMEM((800, 32), jnp.float32),
            pltpu.VMEM((400, 16), jnp.float32),
            pltpu.SemaphoreType.DMA,
        ],
    )
    return f(edge_r, padt, zer1, ones1)


def _agg_body(depth, edge_split, nslabs, t0_ref, t1_ref, edge_ref, spad,
              dpad, zer, p0_ref, p1_ref, acc, sidx, didx, rows, gsem):
    c = lax.axis_index("c")
    s = lax.axis_index("s")
    pltpu.sync_copy(zer, acc.at[pl.ds(s * RPT, RPT)])
    plsc.subcore_barrier()

    wid = c * 16 + s if edge_split else s
    rows_per_w = nslabs * JPS

    lastw = 31 if edge_split else 15

    def run(table):
        @pl.loop(0, nslabs)
        def _slab(sl):
            base = wid * rows_per_w + sl * JPS
            is_last = jnp.logical_and(wid == lastw, sl == nslabs - 1)

            @pl.when(jnp.logical_not(is_last))
            def _():
                pltpu.sync_copy(edge_ref.at[0, pl.ds(base, JPS)], sidx)
                pltpu.sync_copy(edge_ref.at[1, pl.ds(base, JPS)], didx)

            @pl.when(is_last)
            def _():
                pltpu.sync_copy(edge_ref.at[0, pl.ds(6244, 6)],
                                sidx.at[pl.ds(0, 6)])
                pltpu.sync_copy(spad, sidx.at[pl.ds(6, 22)])
                pltpu.sync_copy(edge_ref.at[1, pl.ds(6244, 6)],
                                didx.at[pl.ds(0, 6)])
                pltpu.sync_copy(dpad, didx.at[pl.ds(6, 22)])
            for d in range(depth):
                pltpu.async_copy(table.at[sidx.at[d]], rows.at[d], gsem.at[d])
            for j in range(JPS):
                d = j % depth
                pltpu.make_async_copy(
                    table.at[sidx.at[j]], rows.at[d], gsem.at[d]).wait()
                pltpu.sync_copy(rows.at[d], acc.at[didx.at[j]], add=True)
                if j + depth < JPS:
                    pltpu.async_copy(table.at[sidx.at[j + depth]],
                                     rows.at[d], gsem.at[d])

    if edge_split:
        run(t0_ref)
    else:
        @pl.when(c == 0)
        def _():
            run(t0_ref)

        @pl.when(c == 1)
        def _():
            run(t1_ref)

    plsc.subcore_barrier()
    sl_ = pl.ds(s * RPT, RPT)

    @pl.when(c == 0)
    def _():
        pltpu.sync_copy(acc.at[sl_], p0_ref.at[sl_])

    @pl.when(c == 1)
    def _():
        pltpu.sync_copy(acc.at[sl_], p1_ref.at[sl_])


def _aggregate(table0, table1, edge_r, spad, dpad, zer, feat, edge_split):
    nslabs = 7 if edge_split else 14
    depth = 5 if feat == 32 else 14
    body = functools.partial(_agg_body, depth, edge_split, nslabs)
    f = pl.kernel(
        body,
        out_type=(
            jax.ShapeDtypeStruct((ACC_ROWS, feat), jnp.float32),
            jax.ShapeDtypeStruct((ACC_ROWS, feat), jnp.float32),
        ),
        mesh=_mesh(),
        compiler_params=_SC_PARAMS,
        scratch_types=[
            pltpu.VMEM_SHARED((ACC_ROWS, feat), jnp.float32),
            pltpu.VMEM((JPS, BATCH), jnp.int32),
            pltpu.VMEM((JPS, BATCH), jnp.int32),
            pltpu.VMEM((depth, BATCH, feat), jnp.float32),
            pltpu.SemaphoreType.DMA((depth,)),
        ],
    )
    return f(table0, table1, edge_r, spad, dpad, zer)


def _blk128(nb):
    return pl.BlockSpec((nb, 128), lambda i: (i, 0))


def _wspec():
    return pl.BlockSpec((128, 128), lambda i: (0, 0))


def _bspec():
    return pl.BlockSpec((1, 128), lambda i: (0, 0))


def _inv(deg):
    return lax.rsqrt(jnp.maximum(deg, 1.0))


def _k1_body(x_ref, od_ref, xn_ref):
    xn_ref[...] = x_ref[...] * _inv(od_ref[...][:12500])


def _k1(xpp, od32p):
    return pl.pallas_call(
        _k1_body,
        grid=(1,),
        in_specs=[_blk128(12500), _blk128(12800)],
        out_specs=_blk128(12500),
        out_shape=jax.ShapeDtypeStruct((12500, 128), jnp.float32),
    )(xpp, od32p)


def _k3_body(p0_ref, p1_ref, od_ref, id_ref, wa_ref, wb_ref, ba_ref, bb_ref,
             ha_ref, hb_ref):
    agg = p0_ref[...] + p1_ref[...]
    iin = _inv(id_ref[...])
    iout = _inv(od_ref[...])
    ma = jnp.dot(agg, wa_ref[...], preferred_element_type=jnp.float32)
    mb = jnp.dot(agg, wb_ref[...], preferred_element_type=jnp.float32)
    ha_ref[...] = jnp.tanh(ma * iin + ba_ref[...]) * iout
    hb_ref[...] = jnp.tanh(mb * iin + bb_ref[...]) * iout


def _k3(p0p, p1p, od32p, id32p, w1a, w1b, b1a, b1b):
    return pl.pallas_call(
        _k3_body,
        grid=(2,),
        in_specs=[_blk128(6400), _blk128(6400), _blk128(6400), _blk128(6400),
                  _wspec(), _wspec(), _bspec(), _bspec()],
        out_specs=(_blk128(6400), _blk128(6400)),
        out_shape=(jax.ShapeDtypeStruct((12800, 128), jnp.float32),
                   jax.ShapeDtypeStruct((12800, 128), jnp.float32)),
    )(p0p, p1p, od32p, id32p, w1a, w1b, b1a, b1b)


def _k5_body(a_ref, b_ref, od_ref, id_ref, waa_ref, wba_ref, wab_ref,
             wbb_ref, w3a_ref, w3b_ref, b2a_ref, b2b_ref, g_ref):
    a = a_ref[...]
    b = b_ref[...]
    iin = _inv(id_ref[...])
    iout = _inv(od_ref[...])
    h2a = jnp.tanh((jnp.dot(a, waa_ref[...], preferred_element_type=jnp.float32)
                    + jnp.dot(b, wba_ref[...], preferred_element_type=jnp.float32))
                   * iin + b2a_ref[...]) * iout
    h2b = jnp.tanh((jnp.dot(a, wab_ref[...], preferred_element_type=jnp.float32)
                    + jnp.dot(b, wbb_ref[...], preferred_element_type=jnp.float32))
                   * iin + b2b_ref[...]) * iout
    g_ref[...] = (jnp.dot(h2a, w3a_ref[...], preferred_element_type=jnp.float32)
                  + jnp.dot(h2b, w3b_ref[...], preferred_element_type=jnp.float32))


def _k5(a2ap, a2bp, od32p, id32p, waa, wba, wab, wbb, w3a, w3b, b2a, b2b):
    return pl.pallas_call(
        _k5_body,
        grid=(2,),
        in_specs=[_blk128(6400), _blk128(6400), _blk128(6400), _blk128(6400),
                  _wspec(), _wspec(), _wspec(), _wspec(),
                  _wspec(), _wspec(), _bspec(), _bspec()],
        out_specs=_blk128(6400),
        out_shape=jax.ShapeDtypeStruct((12800, 128), jnp.float32),
    )(a2ap, a2bp, od32p, id32p, waa, wba, wab, wbb, w3a, w3b, b2a, b2b)


def _k7_body(q0_ref, q1_ref, id_ref, b3_ref, o_ref):
    o_ref[...] = (q0_ref[...] + q1_ref[...]) * _inv(id_ref[...]) + b3_ref[...]


def _k7(q0p, q1p, id8p, b3w):
    return pl.pallas_call(
        _k7_body,
        grid=(2,),
        in_specs=[_blk128(1600), _blk128(1600), _blk128(1600), _bspec()],
        out_specs=_blk128(1600),
        out_shape=jax.ShapeDtypeStruct((3200, 128), jnp.float32),
    )(q0p, q1p, id8p, b3w)


def _bd4(m):
    return jnp.kron(jnp.eye(4, dtype=jnp.float32), m)


def kernel(x, edge_index, W1, b1, W2, b2, W3, b3):
    edge_r = edge_index.astype(jnp.int32).reshape(2, 6250, 128)
    pad0 = jnp.zeros((22, 128), jnp.int32)
    padt = jnp.full((22, 128), TRASH, jnp.int32)

    zer1 = jnp.zeros((RPT,), jnp.float32)
    zer32 = jnp.zeros((RPT, 32), jnp.float32)
    ones1 = jnp.ones((BATCH,), jnp.float32)

    od32, id32, id8 = _degrees(edge_r, padt, zer1, ones1)
    od32p = od32.reshape(12800, 128)
    id32p = id32.reshape(12800, 128)
    id8p = id8.reshape(3200, 128)

    xpp = x.reshape(12500, 128)
    xnp = _k1(xpp, od32p)

    xn_t = xnp.reshape(N, 32)
    p0, p1 = _aggregate(xn_t, xn_t, edge_r, pad0, padt, zer32, 32, True)

    w1a = _bd4(W1[:, :32])
    w1b = _bd4(W1[:, 32:])
    b1a = jnp.tile(b1[:32], 4).reshape(1, 128)
    b1b = jnp.tile(b1[32:], 4).reshape(1, 128)
    h1ap, h1bp = _k3(p0.reshape(12800, 128), p1.reshape(12800, 128),
                     od32p, id32p, w1a, w1b, b1a, b1b)

    a2a, a2b = _aggregate(h1ap.reshape(ACC_ROWS, 32), h1bp.reshape(ACC_ROWS, 32),
                          edge_r, pad0, padt, zer32, 32, False)

    w3wide = jnp.pad(W3, ((0, 0), (0, 30)))
    waa = _bd4(W2[:32, :32])
    wba = _bd4(W2[32:, :32])
    wab = _bd4(W2[:32, 32:])
    wbb = _bd4(W2[32:, 32:])
    w3a = _bd4(w3wide[:32])
    w3b = _bd4(w3wide[32:])
    b2a = jnp.tile(b2[:32], 4).reshape(1, 128)
    b2b = jnp.tile(b2[32:], 4).reshape(1, 128)
    gp = _k5(a2a.reshape(12800, 128), a2b.reshape(12800, 128),
             od32p, id32p, waa, wba, wab, wbb, w3a, w3b, b2a, b2b)

    g8 = gp.reshape(12800, 4, 32)[:, :, :8].reshape(ACC_ROWS, 8)
    zer8 = jnp.zeros((RPT, 8), jnp.float32)
    q0, q1 = _aggregate(g8, g8, edge_r, pad0, padt, zer8, 8, True)

    b3w = jnp.tile(jnp.pad(b3, (0, 6)), 16).reshape(1, 128)
    o = _k7(q0.reshape(3200, 128), q1.reshape(3200, 128), id8p, b3w)
    return o.reshape(3200, 16, 8)[:, :, :2].reshape(ACC_ROWS, 2)[:N]


# speedup vs baseline: 1.3543x; 1.0889x over previous
import functools

import jax
import jax.numpy as jnp
from jax import lax
from jax.experimental import pallas as pl
from jax.experimental.pallas import tpu as pltpu
from jax.experimental.pallas import tpu_sc as plsc

N = 50000
E = 800000
EPAD = 802816
ACC_ROWS = 51200
TRASH = N
BATCH = 128
JPS = 28
RPT = 3200


def _mesh():
    return plsc.VectorSubcoreMesh(core_axis_name="c", subcore_axis_name="s")


_SC_PARAMS = pltpu.CompilerParams(use_tc_tiling_on_sc=False)
_SC_PARAMS_NL = pltpu.CompilerParams(use_tc_tiling_on_sc=False,
                                     needs_layout_passes=False)


def _deg_body(edge_ref, padt, zer, ones, od32_ref, id32_ref, id8_ref,
              acc, sidx, ones_v, dbuf, wbuf, wbuf8, dsem):
    c = lax.axis_index("c")
    s = lax.axis_index("s")
    pltpu.sync_copy(zer, acc.at[pl.ds(s * RPT, RPT)])
    pltpu.sync_copy(ones, ones_v)
    plsc.subcore_barrier()

    def hist(row):
        @pl.loop(0, 14)
        def _slab(sl):
            base = s * 392 + sl * JPS
            is_last = jnp.logical_and(s == 15, sl == 13)

            @pl.when(jnp.logical_not(is_last))
            def _():
                pltpu.sync_copy(edge_ref.at[row, pl.ds(base, JPS)], sidx)

            @pl.when(is_last)
            def _():
                pltpu.sync_copy(edge_ref.at[row, pl.ds(6244, 6)],
                                sidx.at[pl.ds(0, 6)])
                pltpu.sync_copy(padt, sidx.at[pl.ds(6, 22)])

            @pl.loop(0, JPS)
            def _j(j):
                pltpu.async_copy(ones_v, acc.at[sidx.at[j]], dsem, add=True)

            @pl.loop(0, JPS)
            def _jw(j):
                pltpu.make_async_copy(ones_v, acc.at[sidx.at[j]], dsem).wait()

    @pl.when(c == 0)
    def _():
        hist(0)

    @pl.when(c == 1)
    def _():
        hist(1)

    plsc.subcore_barrier()

    pltpu.sync_copy(acc.at[pl.ds(s * RPT, RPT)], dbuf)

    def expand_to(out_ref):
        @pl.loop(0, 4)
        def _chunk(k):
            @pl.loop(0, 50)
            def _n16(n16):
                v = dbuf[pl.ds(k * 800 + n16 * 16, 16)]
                for lane in range(16):
                    v16 = jnp.full((16,), v[lane], jnp.float32)
                    wbuf[n16 * 16 + lane, pl.ds(0, 16)] = v16
                    wbuf[n16 * 16 + lane, pl.ds(16, 16)] = v16
            pltpu.sync_copy(wbuf, out_ref.at[pl.ds(s * RPT + k * 800, 800)])

    def expand8_to(out_ref):
        lane = lax.iota(jnp.int32, 16)

        @pl.loop(0, 4)
        def _chunk(k):
            @pl.loop(0, 50)
            def _n16(n16):
                v = dbuf[pl.ds(k * 800 + n16 * 16, 16)]
                for l in range(8):
                    lo = jnp.full((16,), v[2 * l], jnp.float32)
                    hi = jnp.full((16,), v[2 * l + 1], jnp.float32)
                    wbuf8[n16 * 8 + l, :] = jnp.where(lane < 8, lo, hi)
            pltpu.sync_copy(
                wbuf8, out_ref.at[pl.ds(s * 1600 + k * 400, 400)])

    @pl.when(c == 0)
    def _():
        expand_to(od32_ref)

    @pl.when(c == 1)
    def _():
        expand_to(id32_ref)
        expand8_to(id8_ref)


def _degrees(edge_r, padt, zer1, ones1):
    f = pl.kernel(
        _deg_body,
        out_type=(
            jax.ShapeDtypeStruct((ACC_ROWS, 32), jnp.float32),
            jax.ShapeDtypeStruct((ACC_ROWS, 32), jnp.float32),
            jax.ShapeDtypeStruct((ACC_ROWS // 2, 16), jnp.float32),
        ),
        mesh=_mesh(),
        compiler_params=_SC_PARAMS,
        scratch_types=[
            pltpu.VMEM_SHARED((ACC_ROWS,), jnp.float32),
            pltpu.VMEM((JPS, BATCH), jnp.int32),
            pltpu.VMEM((BATCH,), jnp.float32),
            pltpu.VMEM((RPT,), jnp.float32),
            pltpu.VMEM((800, 32), jnp.float32),
            pltpu.VMEM((400, 16), jnp.float32),
            pltpu.SemaphoreType.DMA,
        ],
    )
    return f(edge_r, padt, zer1, ones1)


def _agg_body(depth, edge_split, nslabs, t0_ref, t1_ref, edge_ref, spad,
              dpad, zer, p0_ref, p1_ref, acc, sidx, didx, rows, gsem):
    c = lax.axis_index("c")
    s = lax.axis_index("s")
    pltpu.sync_copy(zer, acc.at[pl.ds(s * RPT, RPT)])
    plsc.subcore_barrier()

    wid = c * 16 + s if edge_split else s
    rows_per_w = nslabs * JPS

    lastw = 31 if edge_split else 15

    def run(table):
        @pl.loop(0, nslabs)
        def _slab(sl):
            base = wid * rows_per_w + sl * JPS
            is_last = jnp.logical_and(wid == lastw, sl == nslabs - 1)

            @pl.when(jnp.logical_not(is_last))
            def _():
                pltpu.sync_copy(edge_ref.at[0, pl.ds(base, JPS)], sidx)
                pltpu.sync_copy(edge_ref.at[1, pl.ds(base, JPS)], didx)

            @pl.when(is_last)
            def _():
                pltpu.sync_copy(edge_ref.at[0, pl.ds(6244, 6)],
                                sidx.at[pl.ds(0, 6)])
                pltpu.sync_copy(spad, sidx.at[pl.ds(6, 22)])
                pltpu.sync_copy(edge_ref.at[1, pl.ds(6244, 6)],
                                didx.at[pl.ds(0, 6)])
                pltpu.sync_copy(dpad, didx.at[pl.ds(6, 22)])
            for d in range(depth):
                pltpu.async_copy(table.at[sidx.at[d]], rows.at[d], gsem.at[d])
            for j in range(JPS):
                d = j % depth
                pltpu.make_async_copy(
                    table.at[sidx.at[j]], rows.at[d], gsem.at[d]).wait()
                pltpu.sync_copy(rows.at[d], acc.at[didx.at[j]], add=True)
                if j + depth < JPS:
                    pltpu.async_copy(table.at[sidx.at[j + depth]],
                                     rows.at[d], gsem.at[d])

    if edge_split:
        run(t0_ref)
    else:
        @pl.when(c == 0)
        def _():
            run(t0_ref)

        @pl.when(c == 1)
        def _():
            run(t1_ref)

    plsc.subcore_barrier()
    sl_ = pl.ds(s * RPT, RPT)

    @pl.when(c == 0)
    def _():
        pltpu.sync_copy(acc.at[sl_], p0_ref.at[sl_])

    @pl.when(c == 1)
    def _():
        pltpu.sync_copy(acc.at[sl_], p1_ref.at[sl_])


def _aggregate(table0, table1, edge_r, spad, dpad, zer, feat, edge_split):
    nslabs = 7 if edge_split else 14
    depth = 5 if feat == 32 else 14
    body = functools.partial(_agg_body, depth, edge_split, nslabs)
    f = pl.kernel(
        body,
        out_type=(
            jax.ShapeDtypeStruct((ACC_ROWS, feat), jnp.float32),
            jax.ShapeDtypeStruct((ACC_ROWS, feat), jnp.float32),
        ),
        mesh=_mesh(),
        compiler_params=_SC_PARAMS,
        scratch_types=[
            pltpu.VMEM_SHARED((ACC_ROWS, feat), jnp.float32),
            pltpu.VMEM((JPS, BATCH), jnp.int32),
            pltpu.VMEM((JPS, BATCH), jnp.int32),
            pltpu.VMEM((depth, BATCH, feat), jnp.float32),
            pltpu.SemaphoreType.DMA((depth,)),
        ],
    )
    return f(table0, table1, edge_r, spad, dpad, zer)


def _agg8_body(t_ref, edge_ref, spad, dpad, zer, q0_ref, q1_ref,
               acc, tbl, sidx, didx, rows, cbuf, cout8, gsem):
    c = lax.axis_index("c")
    s = lax.axis_index("s")
    pltpu.sync_copy(zer, acc.at[pl.ds(s * RPT, RPT)])

    lane = lax.iota(jnp.int32, 16)

    @pl.loop(0, 4)
    def _chunk(k):
        pltpu.sync_copy(t_ref.at[pl.ds(s * RPT + k * 800, 800)], cbuf)

        @pl.loop(0, 400)
        def _i(i):
            p = jnp.full((16,), i * 16, jnp.int32) + lane
            r8 = p >> 3
            c8 = p & 7
            vals = plsc.load_gather(cbuf, [r8, c8])
            plsc.store_scatter(cout8, [r8, c8], vals)
        pltpu.sync_copy(cout8, tbl.at[pl.ds(s * RPT + k * 800, 800)])

    plsc.subcore_barrier()

    wid = c * 16 + s
    depth = 4

    @pl.loop(0, 7)
    def _slab(sl):
        base = wid * 196 + sl * JPS
        is_last = jnp.logical_and(wid == 31, sl == 6)

        @pl.when(jnp.logical_not(is_last))
        def _():
            pltpu.sync_copy(edge_ref.at[0, pl.ds(base, JPS)], sidx)
            pltpu.sync_copy(edge_ref.at[1, pl.ds(base, JPS)], didx)

        @pl.when(is_last)
        def _():
            pltpu.sync_copy(edge_ref.at[0, pl.ds(6244, 6)],
                            sidx.at[pl.ds(0, 6)])
            pltpu.sync_copy(spad, sidx.at[pl.ds(6, 22)])
            pltpu.sync_copy(edge_ref.at[1, pl.ds(6244, 6)],
                            didx.at[pl.ds(0, 6)])
            pltpu.sync_copy(dpad, didx.at[pl.ds(6, 22)])

        for d in range(depth):
            pltpu.async_copy(tbl.at[sidx.at[d]], rows.at[d], gsem.at[d])
        for j in range(JPS):
            d = j % depth
            pltpu.make_async_copy(
                tbl.at[sidx.at[j]], rows.at[d], gsem.at[d]).wait()
            pltpu.sync_copy(rows.at[d], acc.at[didx.at[j]], add=True)
            if j + depth < JPS:
                pltpu.async_copy(tbl.at[sidx.at[j + depth]],
                                 rows.at[d], gsem.at[d])

    plsc.subcore_barrier()
    sl_ = pl.ds(s * RPT, RPT)

    @pl.when(c == 0)
    def _():
        pltpu.sync_copy(acc.at[sl_], q0_ref.at[sl_])

    @pl.when(c == 1)
    def _():
        pltpu.sync_copy(acc.at[sl_], q1_ref.at[sl_])


def _agg8(table, edge_r, spad, dpad, zer):
    f = pl.kernel(
        _agg8_body,
        out_type=(
            jax.ShapeDtypeStruct((ACC_ROWS, 8), jnp.float32),
            jax.ShapeDtypeStruct((ACC_ROWS, 8), jnp.float32),
        ),
        mesh=_mesh(),
        compiler_params=_SC_PARAMS_NL,
        scratch_types=[
            pltpu.VMEM_SHARED((ACC_ROWS, 8), jnp.float32),
            pltpu.VMEM_SHARED((ACC_ROWS, 8), jnp.float32),
            pltpu.VMEM((JPS, BATCH), jnp.int32),
            pltpu.VMEM((JPS, BATCH), jnp.int32),
            pltpu.VMEM((4, BATCH, 8), jnp.float32),
            pltpu.VMEM((800, 32), jnp.float32),
            pltpu.VMEM((800, 8), jnp.float32),
            pltpu.SemaphoreType.DMA((4,)),
        ],
    )
    return f(table, edge_r, spad, dpad, zer)


def _blk128(nb):
    return pl.BlockSpec((nb, 128), lambda i: (i, 0))


def _wspec():
    return pl.BlockSpec((128, 128), lambda i: (0, 0))


def _bspec():
    return pl.BlockSpec((1, 128), lambda i: (0, 0))


def _inv(deg):
    return lax.rsqrt(jnp.maximum(deg, 1.0))


def _k1_body(x_ref, od_ref, xn_ref):
    xn_ref[...] = x_ref[...] * _inv(od_ref[...][:12500])


def _k1(xpp, od32p):
    return pl.pallas_call(
        _k1_body,
        grid=(1,),
        in_specs=[_blk128(12500), _blk128(12800)],
        out_specs=_blk128(12500),
        out_shape=jax.ShapeDtypeStruct((12500, 128), jnp.float32),
    )(xpp, od32p)


def _k3_body(p0_ref, p1_ref, od_ref, id_ref, wa_ref, wb_ref, ba_ref, bb_ref,
             ha_ref, hb_ref):
    agg = p0_ref[...] + p1_ref[...]
    iin = _inv(id_ref[...])
    iout = _inv(od_ref[...])
    ma = jnp.dot(agg, wa_ref[...], preferred_element_type=jnp.float32)
    mb = jnp.dot(agg, wb_ref[...], preferred_element_type=jnp.float32)
    ha_ref[...] = jnp.tanh(ma * iin + ba_ref[...]) * iout
    hb_ref[...] = jnp.tanh(mb * iin + bb_ref[...]) * iout


def _k3(p0p, p1p, od32p, id32p, w1a, w1b, b1a, b1b):
    return pl.pallas_call(
        _k3_body,
        grid=(2,),
        in_specs=[_blk128(6400), _blk128(6400), _blk128(6400), _blk128(6400),
                  _wspec(), _wspec(), _bspec(), _bspec()],
        out_specs=(_blk128(6400), _blk128(6400)),
        out_shape=(jax.ShapeDtypeStruct((12800, 128), jnp.float32),
                   jax.ShapeDtypeStruct((12800, 128), jnp.float32)),
    )(p0p, p1p, od32p, id32p, w1a, w1b, b1a, b1b)


def _k5_body(a_ref, b_ref, od_ref, id_ref, waa_ref, wba_ref, wab_ref,
             wbb_ref, w3a_ref, w3b_ref, b2a_ref, b2b_ref, g_ref):
    a = a_ref[...]
    b = b_ref[...]
    iin = _inv(id_ref[...])
    iout = _inv(od_ref[...])
    h2a = jnp.tanh((jnp.dot(a, waa_ref[...], preferred_element_type=jnp.float32)
                    + jnp.dot(b, wba_ref[...], preferred_element_type=jnp.float32))
                   * iin + b2a_ref[...]) * iout
    h2b = jnp.tanh((jnp.dot(a, wab_ref[...], preferred_element_type=jnp.float32)
                    + jnp.dot(b, wbb_ref[...], preferred_element_type=jnp.float32))
                   * iin + b2b_ref[...]) * iout
    g_ref[...] = (jnp.dot(h2a, w3a_ref[...], preferred_element_type=jnp.float32)
                  + jnp.dot(h2b, w3b_ref[...], preferred_element_type=jnp.float32))


def _k5(a2ap, a2bp, od32p, id32p, waa, wba, wab, wbb, w3a, w3b, b2a, b2b):
    return pl.pallas_call(
        _k5_body,
        grid=(2,),
        in_specs=[_blk128(6400), _blk128(6400), _blk128(6400), _blk128(6400),
                  _wspec(), _wspec(), _wspec(), _wspec(),
                  _wspec(), _wspec(), _bspec(), _bspec()],
        out_specs=_blk128(6400),
        out_shape=jax.ShapeDtypeStruct((12800, 128), jnp.float32),
    )(a2ap, a2bp, od32p, id32p, waa, wba, wab, wbb, w3a, w3b, b2a, b2b)


def _k7_body(q0_ref, q1_ref, id_ref, b3_ref, o_ref):
    o_ref[...] = (q0_ref[...] + q1_ref[...]) * _inv(id_ref[...]) + b3_ref[...]


def _k7(q0p, q1p, id8p, b3w):
    return pl.pallas_call(
        _k7_body,
        grid=(2,),
        in_specs=[_blk128(1600), _blk128(1600), _blk128(1600), _bspec()],
        out_specs=_blk128(1600),
        out_shape=jax.ShapeDtypeStruct((3200, 128), jnp.float32),
    )(q0p, q1p, id8p, b3w)


def _bd4(m):
    return jnp.kron(jnp.eye(4, dtype=jnp.float32), m)


def kernel(x, edge_index, W1, b1, W2, b2, W3, b3):
    edge_r = edge_index.astype(jnp.int32).reshape(2, 6250, 128)
    pad0 = jnp.zeros((22, 128), jnp.int32)
    padt = jnp.full((22, 128), TRASH, jnp.int32)

    zer1 = jnp.zeros((RPT,), jnp.float32)
    zer32 = jnp.zeros((RPT, 32), jnp.float32)
    ones1 = jnp.ones((BATCH,), jnp.float32)

    od32, id32, id8 = _degrees(edge_r, padt, zer1, ones1)
    od32p = od32.reshape(12800, 128)
    id32p = id32.reshape(12800, 128)
    id8p = id8.reshape(3200, 128)

    xpp = x.reshape(12500, 128)
    xnp = _k1(xpp, od32p)

    xn_t = xnp.reshape(N, 32)
    p0, p1 = _aggregate(xn_t, xn_t, edge_r, pad0, padt, zer32, 32, True)

    w1a = _bd4(W1[:, :32])
    w1b = _bd4(W1[:, 32:])
    b1a = jnp.tile(b1[:32], 4).reshape(1, 128)
    b1b = jnp.tile(b1[32:], 4).reshape(1, 128)
    h1ap, h1bp = _k3(p0.reshape(12800, 128), p1.reshape(12800, 128),
                     od32p, id32p, w1a, w1b, b1a, b1b)

    a2a, a2b = _aggregate(h1ap.reshape(ACC_ROWS, 32), h1bp.reshape(ACC_ROWS, 32),
                          edge_r, pad0, padt, zer32, 32, False)

    w3wide = jnp.pad(W3, ((0, 0), (0, 30)))
    waa = _bd4(W2[:32, :32])
    wba = _bd4(W2[32:, :32])
    wab = _bd4(W2[:32, 32:])
    wbb = _bd4(W2[32:, 32:])
    w3a = _bd4(w3wide[:32])
    w3b = _bd4(w3wide[32:])
    b2a = jnp.tile(b2[:32], 4).reshape(1, 128)
    b2b = jnp.tile(b2[32:], 4).reshape(1, 128)
    gp = _k5(a2a.reshape(12800, 128), a2b.reshape(12800, 128),
             od32p, id32p, waa, wba, wab, wbb, w3a, w3b, b2a, b2b)

    zer8 = jnp.zeros((RPT, 8), jnp.float32)
    q0, q1 = _agg8(gp.reshape(ACC_ROWS, 32), edge_r, pad0, padt, zer8)

    b3w = jnp.tile(jnp.pad(b3, (0, 6)), 16).reshape(1, 128)
    o = _k7(q0.reshape(3200, 128), q1.reshape(3200, 128), id8p, b3w)
    return o.reshape(3200, 16, 8)[:, :, :2].reshape(ACC_ROWS, 2)[:N]
